# Initial kernel scaffold; baseline (speedup 1.0000x reference)
import jax
import jax.numpy as jnp
from jax.experimental import pallas as pl


def kernel(x, edge_index, W_lin, b_conv, W1, b1, W2, b2):
    raise NotImplementedError("write your pallas kernel here")


# speedup vs baseline: 13.0150x; 13.0150x over previous
import functools

import jax
import jax.numpy as jnp
from jax import lax
from jax.experimental import pallas as pl
from jax.experimental.pallas import tpu as pltpu
from jax.experimental.pallas import tpu_sc as plsc

N = 10000
NP = 10240
CH = 128
HALF = CH // 2
E = 320000
NS = 16
C = 100
NCHUNK = E // NS // C
RPT = NP // NS
NSUB = 4
SUB = RPT // NSUB
BR = 1000


def _tc_linear(x, w_lin):
    def body(x_ref, w_ref, o0_ref, o1_ref):
        r = lax.dot_general(x_ref[...], w_ref[...], (((1,), (1,)), ((), ())),
                            preferred_element_type=jnp.float32)
        o0_ref[...] = r[:, :HALF]
        o1_ref[...] = r[:, HALF:]

    return pl.pallas_call(
        body,
        grid=(N // BR,),
        in_specs=[pl.BlockSpec((BR, CH), lambda i: (i, 0)),
                  pl.BlockSpec((CH, CH), lambda i: (0, 0))],
        out_specs=[pl.BlockSpec((BR, HALF), lambda i: (i, 0)),
                   pl.BlockSpec((BR, HALF), lambda i: (i, 0))],
        out_shape=[jax.ShapeDtypeStruct((N, HALF), jnp.float32)] * 2,
    )(x, w_lin)


def _tc_mlp(o0, o1, b_conv, w1, b1, w2, b2):
    def body(o0_ref, o1_ref, bc_ref, w1_ref, b1_ref, w2_ref, b2_ref, y_ref):
        t = jnp.concatenate([o0_ref[...], o1_ref[...]], axis=1) + bc_ref[...]
        h = lax.dot_general(t, w1_ref[...], (((1,), (1,)), ((), ())),
                            preferred_element_type=jnp.float32) + b1_ref[...]
        h = jnp.maximum(h, 0.0)
        y = lax.dot_general(h, w2_ref[...], (((1,), (1,)), ((), ())),
                            preferred_element_type=jnp.float32) + b2_ref[...]
        y_ref[...] = jnp.tanh(y)

    return pl.pallas_call(
        body,
        grid=(N // BR,),
        in_specs=[pl.BlockSpec((BR, HALF), lambda i: (i, 0)),
                  pl.BlockSpec((BR, HALF), lambda i: (i, 0)),
                  pl.BlockSpec((1, CH), lambda i: (0, 0)),
                  pl.BlockSpec((CH, CH), lambda i: (0, 0)),
                  pl.BlockSpec((1, CH), lambda i: (0, 0)),
                  pl.BlockSpec((CH, CH), lambda i: (0, 0)),
                  pl.BlockSpec((1, CH), lambda i: (0, 0))],
        out_specs=pl.BlockSpec((BR, CH), lambda i: (i, 0)),
        out_shape=jax.ShapeDtypeStruct((N, CH), jnp.float32),
    )(o0, o1, b_conv.reshape(1, CH), w1, b1.reshape(1, CH), w2,
      b2.reshape(1, CH))


def _sc_conv(xl0, xl1, nidx3, eidx3):
    f32 = jnp.float32
    sds = jax.ShapeDtypeStruct
    mesh = plsc.VectorSubcoreMesh(core_axis_name="c", subcore_axis_name="s",
                                  num_cores=2, num_subcores=NS)

    @functools.partial(
        pl.kernel,
        out_type=(sds((NP, HALF), f32), sds((NP, HALF), f32),
                  sds((NP, HALF), f32), sds((NP, HALF), f32)),
        mesh=mesh,
        scratch_types=[
            pltpu.VMEM_SHARED((NP, HALF), f32),
            pltpu.VMEM_SHARED((NP, 16), f32),
            pltpu.VMEM_SHARED((NP, 16), f32),
            pltpu.VMEM((NCHUNK, C), jnp.int32),
            pltpu.VMEM((NCHUNK, C), jnp.int32),
            pltpu.VMEM((C, HALF), f32),
            pltpu.VMEM((C, 16), f32),
            pltpu.VMEM((SUB, HALF), f32),
            pltpu.VMEM((SUB, 16), f32),
            pltpu.SemaphoreType.DMA,
        ],
        compiler_params=pltpu.CompilerParams(use_tc_tiling_on_sc=False),
    )
    def k(xl0_hbm, xl1_hbm, nidx_hbm, eidx_hbm,
          ef0_hbm, ef1_hbm, out0_hbm, out1_hbm,
          acc_sh, cb_sh, cd_sh,
          eidx_v, nidx_v, rows_v, ones_v, big_v, cnt_v, gsem):
        cid = lax.axis_index("c")
        sid = lax.axis_index("s")
        base = sid * RPT

        def zero_big():
            def z64(r, _):
                for jj in range(HALF // 16):
                    big_v[r, pl.ds(jj * 16, 16)] = jnp.zeros((16,), f32)
                return 0
            lax.fori_loop(0, SUB, z64, 0)

        zero_big()

        def z16(r, _):
            cnt_v[r, :] = jnp.zeros((16,), f32)
            return 0
        lax.fori_loop(0, SUB, z16, 0)

        def o16(r, _):
            ones_v[r, :] = jnp.ones((16,), f32)
            return 0
        lax.fori_loop(0, C, o16, 0)

        for h in range(NSUB):
            hb = base + h * SUB
            pltpu.sync_copy(big_v, acc_sh.at[pl.ds(hb, SUB)])
            pltpu.sync_copy(cnt_v, cb_sh.at[pl.ds(hb, SUB)])
            pltpu.sync_copy(cnt_v, cd_sh.at[pl.ds(hb, SUB)])
        pltpu.sync_copy(eidx_hbm.at[sid], eidx_v)
        pltpu.sync_copy(nidx_hbm.at[sid], nidx_v)
        plsc.subcore_barrier()

        def run_pass1(xl_hbm):
            def body(j, _):
                nr = nidx_v.at[j]
                er = eidx_v.at[j]
                pltpu.async_copy(xl_hbm.at[nr], rows_v, gsem).wait()
                pltpu.sync_copy(rows_v, acc_sh.at[er], add=True)
                pltpu.sync_copy(ones_v, cb_sh.at[er], add=True)
                pltpu.sync_copy(ones_v, cd_sh.at[nr], add=True)
                return 0
            lax.fori_loop(0, NCHUNK, body, 0)

        @pl.when(cid == 0)
        def _():
            run_pass1(xl0_hbm)

        @pl.when(cid == 1)
        def _():
            run_pass1(xl1_hbm)

        plsc.subcore_barrier()

        def scale_body(r, _):
            c = cnt_v[r, :]
            s = jnp.where(c > 0.0, 1.0 / c, 0.0)
            for jj in range(HALF // 16):
                sl = pl.ds(jj * 16, 16)
                big_v[r, sl] = big_v[r, sl] * s
            return 0

        for h in range(NSUB):
            hb = base + h * SUB
            pltpu.sync_copy(acc_sh.at[pl.ds(hb, SUB)], big_v)
            pltpu.sync_copy(cb_sh.at[pl.ds(hb, SUB)], cnt_v)
            lax.fori_loop(0, SUB, scale_body, 0)

            @pl.when(cid == 0)
            def _():
                pltpu.sync_copy(big_v, ef0_hbm.at[pl.ds(hb, SUB)])

            @pl.when(cid == 1)
            def _():
                pltpu.sync_copy(big_v, ef1_hbm.at[pl.ds(hb, SUB)])

        zero_big()
        for h in range(NSUB):
            hb = base + h * SUB
            pltpu.sync_copy(big_v, acc_sh.at[pl.ds(hb, SUB)])
        plsc.subcore_barrier()

        def run_pass2(ef_hbm):
            def body(j, _):
                nr = nidx_v.at[j]
                er = eidx_v.at[j]
                pltpu.async_copy(ef_hbm.at[er], rows_v, gsem).wait()
                pltpu.sync_copy(rows_v, acc_sh.at[nr], add=True)
                return 0
            lax.fori_loop(0, NCHUNK, body, 0)

        @pl.when(cid == 0)
        def _():
            run_pass2(ef0_hbm)

        @pl.when(cid == 1)
        def _():
            run_pass2(ef1_hbm)

        plsc.subcore_barrier()

        for h in range(NSUB):
            hb = base + h * SUB
            pltpu.sync_copy(acc_sh.at[pl.ds(hb, SUB)], big_v)
            pltpu.sync_copy(cd_sh.at[pl.ds(hb, SUB)], cnt_v)
            lax.fori_loop(0, SUB, scale_body, 0)

            @pl.when(cid == 0)
            def _():
                pltpu.sync_copy(big_v, out0_hbm.at[pl.ds(hb, SUB)])

            @pl.when(cid == 1)
            def _():
                pltpu.sync_copy(big_v, out1_hbm.at[pl.ds(hb, SUB)])

    return k(xl0, xl1, nidx3, eidx3)


def kernel(x, edge_index, W_lin, b_conv, W1, b1, W2, b2):
    nidx = edge_index[0].astype(jnp.int32).reshape(NS, NCHUNK, C)
    eidx = edge_index[1].astype(jnp.int32).reshape(NS, NCHUNK, C)
    xl0, xl1 = _tc_linear(x, W_lin)
    _, _, o0, o1 = _sc_conv(xl0, xl1, nidx, eidx)
    return _tc_mlp(o0, o1, b_conv, W1, b1, W2, b2)


# speedup vs baseline: 22.0070x; 1.6909x over previous
import functools

import jax
import jax.numpy as jnp
from jax import lax
from jax.experimental import pallas as pl
from jax.experimental.pallas import tpu as pltpu
from jax.experimental.pallas import tpu_sc as plsc

N = 10000
NP = 10240
CH = 128
HALF = CH // 2
E = 320000
NS = 16
C = 100
NCHUNK = E // NS // C
RPT = NP // NS
NSUB = 5
SUB = RPT // NSUB
BR = 1000


def _tc_linear(x, w_lin):
    def body(x_ref, w_ref, o0_ref, o1_ref):
        r = lax.dot_general(x_ref[...], w_ref[...], (((1,), (1,)), ((), ())),
                            preferred_element_type=jnp.float32)
        o0_ref[...] = r[:, :HALF]
        o1_ref[...] = r[:, HALF:]

    return pl.pallas_call(
        body,
        grid=(N // BR,),
        in_specs=[pl.BlockSpec((BR, CH), lambda i: (i, 0)),
                  pl.BlockSpec((CH, CH), lambda i: (0, 0))],
        out_specs=[pl.BlockSpec((BR, HALF), lambda i: (i, 0)),
                   pl.BlockSpec((BR, HALF), lambda i: (i, 0))],
        out_shape=[jax.ShapeDtypeStruct((N, HALF), jnp.float32)] * 2,
    )(x, w_lin)


def _tc_mlp(o0, o1, b_conv, w1, b1, w2, b2):
    def body(o0_ref, o1_ref, bc_ref, w1_ref, b1_ref, w2_ref, b2_ref, y_ref):
        t = jnp.concatenate([o0_ref[...], o1_ref[...]], axis=1) + bc_ref[...]
        h = lax.dot_general(t, w1_ref[...], (((1,), (1,)), ((), ())),
                            preferred_element_type=jnp.float32) + b1_ref[...]
        h = jnp.maximum(h, 0.0)
        y = lax.dot_general(h, w2_ref[...], (((1,), (1,)), ((), ())),
                            preferred_element_type=jnp.float32) + b2_ref[...]
        y_ref[...] = jnp.tanh(y)

    return pl.pallas_call(
        body,
        grid=(N // BR,),
        in_specs=[pl.BlockSpec((BR, HALF), lambda i: (i, 0)),
                  pl.BlockSpec((BR, HALF), lambda i: (i, 0)),
                  pl.BlockSpec((1, CH), lambda i: (0, 0)),
                  pl.BlockSpec((CH, CH), lambda i: (0, 0)),
                  pl.BlockSpec((1, CH), lambda i: (0, 0)),
                  pl.BlockSpec((CH, CH), lambda i: (0, 0)),
                  pl.BlockSpec((1, CH), lambda i: (0, 0))],
        out_specs=pl.BlockSpec((BR, CH), lambda i: (i, 0)),
        out_shape=jax.ShapeDtypeStruct((N, CH), jnp.float32),
    )(o0, o1, b_conv.reshape(1, CH), w1, b1.reshape(1, CH), w2,
      b2.reshape(1, CH))


def _sc_conv(xl0, xl1, nidx3, eidx3):
    f32 = jnp.float32
    sds = jax.ShapeDtypeStruct
    mesh = plsc.VectorSubcoreMesh(core_axis_name="c", subcore_axis_name="s",
                                  num_cores=2, num_subcores=NS)

    @functools.partial(
        pl.kernel,
        out_type=(sds((NP, HALF), f32), sds((NP, HALF), f32),
                  sds((NP, HALF), f32), sds((NP, HALF), f32)),
        mesh=mesh,
        scratch_types=[
            pltpu.VMEM_SHARED((NP, HALF), f32),
            pltpu.VMEM_SHARED((NP, 16), f32),
            pltpu.VMEM_SHARED((NP, 16), f32),
            pltpu.VMEM((NCHUNK, C), jnp.int32),
            pltpu.VMEM((NCHUNK, C), jnp.int32),
            pltpu.VMEM((C, HALF), f32),
            pltpu.VMEM((C, HALF), f32),
            pltpu.VMEM((C, 16), f32),
            pltpu.VMEM((SUB, HALF), f32),
            pltpu.VMEM((SUB, 16), f32),
            pltpu.SemaphoreType.DMA,
            pltpu.SemaphoreType.DMA,
            pltpu.SemaphoreType.DMA,
            pltpu.SemaphoreType.DMA,
            pltpu.SemaphoreType.DMA,
        ],
        compiler_params=pltpu.CompilerParams(use_tc_tiling_on_sc=False),
    )
    def k(xl0_hbm, xl1_hbm, nidx_hbm, eidx_hbm,
          ef0_hbm, ef1_hbm, out0_hbm, out1_hbm,
          acc_sh, cb_sh, cd_sh,
          eidx_v, nidx_v, rows0, rows1, ones_v, big_v, cnt_v,
          gsem0, gsem1, ssem0, ssem1, csem):
        cid = lax.axis_index("c")
        sid = lax.axis_index("s")
        base = sid * RPT

        def zero_big():
            def z64(r, _):
                for jj in range(HALF // 16):
                    big_v[r, pl.ds(jj * 16, 16)] = jnp.zeros((16,), f32)
                return 0
            lax.fori_loop(0, SUB, z64, 0)

        zero_big()

        def z16(r, _):
            cnt_v[r, :] = jnp.zeros((16,), f32)
            return 0
        lax.fori_loop(0, SUB, z16, 0)

        def o16(r, _):
            ones_v[r, :] = jnp.ones((16,), f32)
            return 0
        lax.fori_loop(0, C, o16, 0)

        for h in range(NSUB):
            hb = base + h * SUB
            pltpu.sync_copy(big_v, acc_sh.at[pl.ds(hb, SUB)])
            pltpu.sync_copy(cnt_v, cb_sh.at[pl.ds(hb, SUB)])
            pltpu.sync_copy(cnt_v, cd_sh.at[pl.ds(hb, SUB)])
        pltpu.sync_copy(eidx_hbm.at[sid], eidx_v)
        pltpu.sync_copy(nidx_hbm.at[sid], nidx_v)
        plsc.subcore_barrier()

        def run_pass(table_hbm, src_idx, dst_idx, with_counts):
            bufs = ((rows0, gsem0, ssem0), (rows1, gsem1, ssem1))
            for b in range(2):
                rb, gs, _ = bufs[b]
                pltpu.async_copy(table_hbm.at[src_idx.at[b]], rb, gs)

            def body(jj, _):
                for b in range(2):
                    rb, gs, ss = bufs[b]
                    j = jj * 2 + b
                    pltpu.make_async_copy(
                        table_hbm.at[src_idx.at[j]], rb, gs).wait()
                    pltpu.async_copy(rb, acc_sh.at[dst_idx.at[j]], ss,
                                     add=True)
                    if with_counts:
                        pltpu.async_copy(ones_v, cb_sh.at[eidx_v.at[j]],
                                         csem, add=True)
                        pltpu.async_copy(ones_v, cd_sh.at[nidx_v.at[j]],
                                         csem, add=True)
                    pltpu.make_async_copy(
                        rb, acc_sh.at[dst_idx.at[j]], ss).wait()

                    @pl.when(j + 2 < NCHUNK)
                    def _():
                        pltpu.async_copy(
                            table_hbm.at[src_idx.at[j + 2]], rb, gs)
                return 0
            lax.fori_loop(0, NCHUNK // 2, body, 0)

            if with_counts:
                def drain(i, _):
                    pltpu.make_async_copy(ones_v, cb_sh.at[eidx_v.at[0]],
                                          csem).wait()
                    return 0
                lax.fori_loop(0, 2 * NCHUNK, drain, 0)

        @pl.when(cid == 0)
        def _():
            run_pass(xl0_hbm, nidx_v, eidx_v, True)

        @pl.when(cid == 1)
        def _():
            run_pass(xl1_hbm, nidx_v, eidx_v, True)

        plsc.subcore_barrier()

        def scale_body(r, _):
            c = cnt_v[r, :]
            s = jnp.where(c > 0.0, 1.0 / c, 0.0)
            for jj in range(HALF // 16):
                sl = pl.ds(jj * 16, 16)
                big_v[r, sl] = big_v[r, sl] * s
            return 0

        for h in range(NSUB):
            hb = base + h * SUB
            pltpu.sync_copy(acc_sh.at[pl.ds(hb, SUB)], big_v)
            pltpu.sync_copy(cb_sh.at[pl.ds(hb, SUB)], cnt_v)
            lax.fori_loop(0, SUB, scale_body, 0)

            @pl.when(cid == 0)
            def _():
                pltpu.sync_copy(big_v, ef0_hbm.at[pl.ds(hb, SUB)])

            @pl.when(cid == 1)
            def _():
                pltpu.sync_copy(big_v, ef1_hbm.at[pl.ds(hb, SUB)])

        zero_big()
        for h in range(NSUB):
            hb = base + h * SUB
            pltpu.sync_copy(big_v, acc_sh.at[pl.ds(hb, SUB)])
        plsc.subcore_barrier()

        @pl.when(cid == 0)
        def _():
            run_pass(ef0_hbm, eidx_v, nidx_v, False)

        @pl.when(cid == 1)
        def _():
            run_pass(ef1_hbm, eidx_v, nidx_v, False)

        plsc.subcore_barrier()

        for h in range(NSUB):
            hb = base + h * SUB
            pltpu.sync_copy(acc_sh.at[pl.ds(hb, SUB)], big_v)
            pltpu.sync_copy(cd_sh.at[pl.ds(hb, SUB)], cnt_v)
            lax.fori_loop(0, SUB, scale_body, 0)

            @pl.when(cid == 0)
            def _():
                pltpu.sync_copy(big_v, out0_hbm.at[pl.ds(hb, SUB)])

            @pl.when(cid == 1)
            def _():
                pltpu.sync_copy(big_v, out1_hbm.at[pl.ds(hb, SUB)])

    return k(xl0, xl1, nidx3, eidx3)


def kernel(x, edge_index, W_lin, b_conv, W1, b1, W2, b2):
    nidx = edge_index[0].astype(jnp.int32).reshape(NS, NCHUNK, C)
    eidx = edge_index[1].astype(jnp.int32).reshape(NS, NCHUNK, C)
    xl0, xl1 = _tc_linear(x, W_lin)
    _, _, o0, o1 = _sc_conv(xl0, xl1, nidx, eidx)
    return _tc_mlp(o0, o1, b_conv, W1, b1, W2, b2)
